# baseline (speedup 1.0000x reference)
import functools

import jax
import jax.numpy as jnp
from jax import lax
from jax.experimental import pallas as pl
from jax.experimental.pallas import tpu as pltpu
from jax.experimental.pallas import tpu_sc as plsc

_ROWS = 32
_L = 16


_NW = 32


def _sc_gather_body(t_hbm, sac_hbm, somac_hbm, c1_hbm, c2_hbm,
                    idx_v, o1_v, o2_v, sem):
    per = t_hbm.shape[0] // _NW
    wid = lax.axis_index("s") * 2 + lax.axis_index("c")
    base = wid * per
    pltpu.sync_copy(t_hbm.at[pl.ds(base, per)], idx_v)
    pltpu.async_copy(sac_hbm.at[idx_v], o1_v, sem).wait()
    pltpu.async_copy(somac_hbm.at[idx_v], o2_v, sem).wait()
    pltpu.sync_copy(o1_v, c1_hbm.at[pl.ds(base, per)])
    pltpu.sync_copy(o2_v, c2_hbm.at[pl.ds(base, per)])


def _sc_gather(t, sac, somac):
    B = t.shape[0]
    per = B // _NW
    mesh = plsc.VectorSubcoreMesh(core_axis_name="c", subcore_axis_name="s")
    f = functools.partial(
        pl.kernel,
        mesh=mesh,
        out_type=[jax.ShapeDtypeStruct((B,), jnp.float32)] * 2,
        scratch_types=[
            pltpu.VMEM((per,), jnp.int32),
            pltpu.VMEM((per,), jnp.float32),
            pltpu.VMEM((per,), jnp.float32),
            pltpu.SemaphoreType.DMA,
        ],
    )(_sc_gather_body)
    return f(t, sac, somac)


def _lerp_body(c1_ref, c2_ref, x_ref, n_ref, o_ref):
    o_ref[...] = c1_ref[...] * x_ref[...] + c2_ref[...] * n_ref[...]


def kernel(x_start, t, noise, sqrt_alphas_cumprod, sqrt_one_minus_alphas_cumprod):
    B = x_start.shape[0]
    F = x_start.size // B
    c1, c2 = _sc_gather(t, sqrt_alphas_cumprod, sqrt_one_minus_alphas_cumprod)
    xf = x_start.reshape(B, F)
    nf = noise.reshape(B, F)
    grid = (B // _ROWS,)
    out = pl.pallas_call(
        _lerp_body,
        grid=grid,
        in_specs=[
            pl.BlockSpec((_ROWS, 1), lambda i: (i, 0)),
            pl.BlockSpec((_ROWS, 1), lambda i: (i, 0)),
            pl.BlockSpec((_ROWS, F), lambda i: (i, 0)),
            pl.BlockSpec((_ROWS, F), lambda i: (i, 0)),
        ],
        out_specs=pl.BlockSpec((_ROWS, F), lambda i: (i, 0)),
        out_shape=jax.ShapeDtypeStruct((B, F), jnp.float32),
    )(c1.reshape(B, 1), c2.reshape(B, 1), xf, nf)
    return out.reshape(x_start.shape)


# speedup vs baseline: 1.2432x; 1.2432x over previous
import jax
import jax.numpy as jnp
from jax.experimental import pallas as pl
from jax.experimental.pallas import tpu as pltpu

_ROWS = 32
_TPAD = 1024


def _lerp_body(t_ref, sac_ref, somac_ref, x_ref, n_ref, o_ref):
    rows = t_ref.shape[0]
    lane = jax.lax.broadcasted_iota(jnp.int32, (rows, _TPAD), 1)
    hot = lane == t_ref[...]
    zero = jnp.zeros((rows, _TPAD), jnp.float32)
    c1 = jnp.sum(jnp.where(hot, sac_ref[...], zero), axis=1, keepdims=True)
    c2 = jnp.sum(jnp.where(hot, somac_ref[...], zero), axis=1, keepdims=True)
    o_ref[...] = c1 * x_ref[...] + c2 * n_ref[...]


def kernel(x_start, t, noise, sqrt_alphas_cumprod, sqrt_one_minus_alphas_cumprod):
    B = x_start.shape[0]
    F = x_start.size // B
    xf = x_start.reshape(B, F)
    nf = noise.reshape(B, F)
    t2 = t.reshape(B, 1)
    sac = jnp.pad(sqrt_alphas_cumprod, (0, _TPAD - sqrt_alphas_cumprod.shape[0]))
    somac = jnp.pad(
        sqrt_one_minus_alphas_cumprod,
        (0, _TPAD - sqrt_one_minus_alphas_cumprod.shape[0]),
    ).reshape(1, _TPAD)
    sac = sac.reshape(1, _TPAD)

    grid = (B // _ROWS,)
    out = pl.pallas_call(
        _lerp_body,
        grid=grid,
        in_specs=[
            pl.BlockSpec((_ROWS, 1), lambda i: (i, 0)),
            pl.BlockSpec((1, _TPAD), lambda i: (0, 0)),
            pl.BlockSpec((1, _TPAD), lambda i: (0, 0)),
            pl.BlockSpec((_ROWS, F), lambda i: (i, 0)),
            pl.BlockSpec((_ROWS, F), lambda i: (i, 0)),
        ],
        out_specs=pl.BlockSpec((_ROWS, F), lambda i: (i, 0)),
        out_shape=jax.ShapeDtypeStruct((B, F), jnp.float32),
    )(t2, sac, somac, xf, nf)
    return out.reshape(x_start.shape)
